# baseline (speedup 1.0000x reference)
import numpy as np
import jax
import jax.numpy as jnp
from jax.experimental import pallas as pl
from jax.experimental.pallas import tpu as pltpu

_TEMPERATURE = 0.07
_MULT = 1.0

_B = 16
_N_POS = 4
_N_NEG = 32
_M = _N_POS + _N_NEG
_Q = _B * _M
_P = 100000
_D = 128

_H = 2
_TP = 10000
_NT = _P // (_H * _TP)


def _max_body(xT_ref, p_ref, out_ref, acc_ref):
    i = pl.program_id(1)

    p = p_ref[...]
    pnsq = jnp.sum(p * p, axis=1, keepdims=True)
    ps = p * jax.lax.rsqrt(pnsq + 1e-37)
    dots = jax.lax.dot_general(
        ps.astype(jnp.bfloat16), xT_ref[...], (((1,), (0,)), ((), ())),
        preferred_element_type=jnp.float32)
    tmax = jnp.max(dots.reshape(_TP // 8, 8, _Q), axis=0)

    @pl.when(i == 0)
    def _init():
        acc_ref[...] = tmax

    @pl.when(i > 0)
    def _accum():
        acc_ref[...] = jnp.maximum(acc_ref[...], tmax)

    @pl.when(i == _NT - 1)
    def _emit():
        out_ref[...] = acc_ref[...]


def _loss_body(macc_ref, xT_ref, tcp_ref, tcnf_ref, sposT_ref, snegT_ref,
               out_ref):
    xT = xT_ref[...]
    xn = jnp.sqrt(jnp.sum(xT * xT, axis=0, keepdims=True))
    max_dot = jnp.max(macc_ref[...], axis=0, keepdims=True)
    max_sim = max_dot * (1.0 / jnp.maximum(xn, 1e-30))
    mins = 1.0 - max_sim

    pv_sum = jax.lax.dot_general(
        mins, sposT_ref[...], (((1,), (0,)), ((), ())),
        preferred_element_type=jnp.float32)
    pv = (pv_sum * _MULT + tcp_ref[...]) / _TEMPERATURE

    nv = (mins * _MULT + tcnf_ref[...]) / _TEMPERATURE
    e = jnp.exp(-nv)
    den = jax.lax.dot_general(
        e, snegT_ref[...], (((1,), (0,)), ((), ())),
        preferred_element_type=jnp.float32)

    num = jnp.exp(-pv)
    losses = -jnp.log(num / (num + den))
    out_ref[...] = jnp.sum(losses, keepdims=True) / _B


def kernel(positive_embs, negative_embs, purch_embeddings,
           target_cost_pos_react, target_cost_neg_react):
    x = jnp.concatenate([positive_embs, negative_embs], axis=1)
    xTf = x.reshape(_Q, _D).T
    xT = xTf.astype(jnp.bfloat16)

    tcp = target_cost_pos_react.reshape(1, _B).astype(jnp.float32)
    tcnf = jnp.concatenate(
        [jnp.zeros((_B, _N_POS), jnp.float32), target_cost_neg_react],
        axis=1).reshape(1, _Q)

    r = np.arange(_Q)
    b = r // _M
    j = r % _M
    onehot = (b[:, None] == np.arange(_B)[None, :])
    sposT = (onehot & (j < _N_POS)[:, None]).astype(np.float32)
    snegT = (onehot & (j >= _N_POS)[:, None]).astype(np.float32)

    macc = pl.pallas_call(
        _max_body,
        grid=(_H, _NT),
        in_specs=[
            pl.BlockSpec((_D, _Q), lambda h, i: (0, 0)),
            pl.BlockSpec((_TP, _D), lambda h, i: (h * _NT + i, 0)),
        ],
        out_specs=pl.BlockSpec((8, _Q), lambda h, i: (h, 0)),
        out_shape=jax.ShapeDtypeStruct((8 * _H, _Q), jnp.float32),
        scratch_shapes=[pltpu.VMEM((8, _Q), jnp.float32)],
        compiler_params=pltpu.CompilerParams(
            dimension_semantics=("parallel", "arbitrary"),
            vmem_limit_bytes=120 * 1024 * 1024),
    )(xT, purch_embeddings)

    out = pl.pallas_call(
        _loss_body,
        in_specs=[
            pl.BlockSpec((8 * _H, _Q), lambda: (0, 0)),
            pl.BlockSpec((_D, _Q), lambda: (0, 0)),
            pl.BlockSpec((1, _B), lambda: (0, 0)),
            pl.BlockSpec((1, _Q), lambda: (0, 0)),
            pl.BlockSpec((_Q, _B), lambda: (0, 0)),
            pl.BlockSpec((_Q, _B), lambda: (0, 0)),
        ],
        out_specs=pl.BlockSpec((1, 1), lambda: (0, 0)),
        out_shape=jax.ShapeDtypeStruct((1, 1), jnp.float32),
    )(macc, xTf, tcp, tcnf, jnp.asarray(sposT), jnp.asarray(snegT))
    return out.reshape(())


# speedup vs baseline: 1.1846x; 1.1846x over previous
import numpy as np
import jax
import jax.numpy as jnp
from jax.experimental import pallas as pl
from jax.experimental.pallas import tpu as pltpu

_TEMPERATURE = 0.07
_MULT = 1.0

_B = 16
_N_POS = 4
_N_NEG = 32
_M = _N_POS + _N_NEG
_Q = _B * _M
_P = 100000
_D = 128

_TP = 20000
_NT = _P // _TP


def _body(xT_ref, p_ref, tcp_ref, tcnf_ref, sposT_ref, snegT_ref,
          out_ref, acc_ref):
    i = pl.program_id(0)

    p = p_ref[...]
    pnsq = jnp.sum(p * p, axis=1, keepdims=True)
    ps = p * jax.lax.rsqrt(pnsq + 1e-37)
    dots = jax.lax.dot_general(
        ps.astype(jnp.bfloat16), xT_ref[...], (((1,), (0,)), ((), ())),
        preferred_element_type=jnp.float32)
    tmax = jnp.max(dots.reshape(_TP // 8, 8, _Q), axis=0)

    @pl.when(i == 0)
    def _init():
        acc_ref[...] = tmax

    @pl.when(i > 0)
    def _accum():
        acc_ref[...] = jnp.maximum(acc_ref[...], tmax)

    @pl.when(i == _NT - 1)
    def _epilogue():
        xT = xT_ref[...].astype(jnp.float32)
        xn = jnp.sqrt(jnp.sum(xT * xT, axis=0, keepdims=True))
        max_sim = (jnp.max(acc_ref[...], axis=0, keepdims=True)
                   * (1.0 / jnp.maximum(xn, 1e-30)))
        mins = 1.0 - max_sim

        pv_sum = jax.lax.dot_general(
            mins, sposT_ref[...], (((1,), (0,)), ((), ())),
            preferred_element_type=jnp.float32)
        pv = (pv_sum * _MULT + tcp_ref[...]) / _TEMPERATURE

        nv = (mins * _MULT + tcnf_ref[...]) / _TEMPERATURE
        e = jnp.exp(-nv)
        den = jax.lax.dot_general(
            e, snegT_ref[...], (((1,), (0,)), ((), ())),
            preferred_element_type=jnp.float32)

        num = jnp.exp(-pv)
        losses = -jnp.log(num / (num + den))
        out_ref[...] = jnp.sum(losses, keepdims=True) / _B


def kernel(positive_embs, negative_embs, purch_embeddings,
           target_cost_pos_react, target_cost_neg_react):
    x = jnp.concatenate([positive_embs, negative_embs], axis=1)
    xT = x.reshape(_Q, _D).T.astype(jnp.bfloat16)

    tcp = target_cost_pos_react.reshape(1, _B).astype(jnp.float32)
    tcnf = jnp.concatenate(
        [jnp.zeros((_B, _N_POS), jnp.float32), target_cost_neg_react],
        axis=1).reshape(1, _Q)

    r = np.arange(_Q)
    b = r // _M
    j = r % _M
    onehot = (b[:, None] == np.arange(_B)[None, :])
    sposT = (onehot & (j < _N_POS)[:, None]).astype(np.float32)
    snegT = (onehot & (j >= _N_POS)[:, None]).astype(np.float32)

    out = pl.pallas_call(
        _body,
        grid=(_NT,),
        in_specs=[
            pl.BlockSpec((_D, _Q), lambda i: (0, 0)),
            pl.BlockSpec((_TP, _D), lambda i: (i, 0)),
            pl.BlockSpec((1, _B), lambda i: (0, 0)),
            pl.BlockSpec((1, _Q), lambda i: (0, 0)),
            pl.BlockSpec((_Q, _B), lambda i: (0, 0)),
            pl.BlockSpec((_Q, _B), lambda i: (0, 0)),
        ],
        out_specs=pl.BlockSpec((1, 1), lambda i: (0, 0)),
        out_shape=jax.ShapeDtypeStruct((1, 1), jnp.float32),
        scratch_shapes=[pltpu.VMEM((8, _Q), jnp.float32)],
        compiler_params=pltpu.CompilerParams(
            dimension_semantics=("arbitrary",),
            vmem_limit_bytes=120 * 1024 * 1024),
    )(xT, purch_embeddings, tcp, tcnf, jnp.asarray(sposT), jnp.asarray(snegT))
    return out.reshape(())


# speedup vs baseline: 1.1857x; 1.0009x over previous
import numpy as np
import jax
import jax.numpy as jnp
from jax.experimental import pallas as pl
from jax.experimental.pallas import tpu as pltpu

_TEMPERATURE = 0.07
_MULT = 1.0

_B = 16
_N_POS = 4
_N_NEG = 32
_M = _N_POS + _N_NEG
_Q = _B * _M
_P = 100000
_D = 128

_TP = 20000
_NT = _P // _TP


def _body(xT_ref, p_ref, tcp_ref, tcnf_ref, sposT_ref, snegT_ref,
          out_ref, acc_ref):
    i = pl.program_id(0)

    p = p_ref[...]
    pnsq = jnp.sum(p * p, axis=1, keepdims=True)
    ps = p * jax.lax.rsqrt(pnsq + 1e-37)
    dots = jax.lax.dot_general(
        ps.astype(jnp.bfloat16), xT_ref[...], (((1,), (0,)), ((), ())),
        preferred_element_type=jnp.float32)
    half = _TP // 2
    m0 = jnp.max(dots[:half].reshape(half // 8, 8, _Q), axis=0)
    m1 = jnp.max(dots[half:].reshape(half // 8, 8, _Q), axis=0)
    tmax = jnp.maximum(m0, m1)

    @pl.when(i == 0)
    def _init():
        acc_ref[...] = tmax

    @pl.when(i > 0)
    def _accum():
        acc_ref[...] = jnp.maximum(acc_ref[...], tmax)

    @pl.when(i == _NT - 1)
    def _epilogue():
        xT = xT_ref[...].astype(jnp.float32)
        xn = jnp.sqrt(jnp.sum(xT * xT, axis=0, keepdims=True))
        max_sim = (jnp.max(acc_ref[...], axis=0, keepdims=True)
                   * (1.0 / jnp.maximum(xn, 1e-30)))
        mins = 1.0 - max_sim

        pv_sum = jax.lax.dot_general(
            mins, sposT_ref[...], (((1,), (0,)), ((), ())),
            preferred_element_type=jnp.float32)
        pv = (pv_sum * _MULT + tcp_ref[...]) / _TEMPERATURE

        nv = (mins * _MULT + tcnf_ref[...]) / _TEMPERATURE
        e = jnp.exp(-nv)
        den = jax.lax.dot_general(
            e, snegT_ref[...], (((1,), (0,)), ((), ())),
            preferred_element_type=jnp.float32)

        num = jnp.exp(-pv)
        losses = -jnp.log(num / (num + den))
        out_ref[...] = jnp.sum(losses, keepdims=True) / _B


def kernel(positive_embs, negative_embs, purch_embeddings,
           target_cost_pos_react, target_cost_neg_react):
    x = jnp.concatenate([positive_embs, negative_embs], axis=1)
    xT = x.reshape(_Q, _D).T.astype(jnp.bfloat16)

    tcp = target_cost_pos_react.reshape(1, _B).astype(jnp.float32)
    tcnf = jnp.concatenate(
        [jnp.zeros((_B, _N_POS), jnp.float32), target_cost_neg_react],
        axis=1).reshape(1, _Q)

    r = np.arange(_Q)
    b = r // _M
    j = r % _M
    onehot = (b[:, None] == np.arange(_B)[None, :])
    sposT = (onehot & (j < _N_POS)[:, None]).astype(np.float32)
    snegT = (onehot & (j >= _N_POS)[:, None]).astype(np.float32)

    out = pl.pallas_call(
        _body,
        grid=(_NT,),
        in_specs=[
            pl.BlockSpec((_D, _Q), lambda i: (0, 0)),
            pl.BlockSpec((_TP, _D), lambda i: (i, 0)),
            pl.BlockSpec((1, _B), lambda i: (0, 0)),
            pl.BlockSpec((1, _Q), lambda i: (0, 0)),
            pl.BlockSpec((_Q, _B), lambda i: (0, 0)),
            pl.BlockSpec((_Q, _B), lambda i: (0, 0)),
        ],
        out_specs=pl.BlockSpec((1, 1), lambda i: (0, 0)),
        out_shape=jax.ShapeDtypeStruct((1, 1), jnp.float32),
        scratch_shapes=[pltpu.VMEM((8, _Q), jnp.float32)],
        compiler_params=pltpu.CompilerParams(
            dimension_semantics=("arbitrary",),
            vmem_limit_bytes=120 * 1024 * 1024),
    )(xT, purch_embeddings, tcp, tcnf, jnp.asarray(sposT), jnp.asarray(snegT))
    return out.reshape(())
